# baseline (speedup 1.0000x reference)
import functools

import numpy as np

import jax
import jax.numpy as jnp
from jax import lax
from jax.experimental import pallas as pl
from jax.experimental.pallas import tpu as pltpu
from jax.experimental.pallas import tpu_sc as plsc

NC = 2
NS = 16

SPW = 8
SUBB = 128


def kernel(sentence, word_embeddings):
    B, S = sentence.shape
    D = word_embeddings.shape[1]
    n = B * S
    n_act = S // SPW
    n_sub = B // SUBB
    n_qp = n_sub // 2
    fg_n = D // 8
    bb_sub = SUBB // 128
    tz = bb_sub * 8 * 128
    tlen = fg_n * tz

    idx = sentence.T.reshape(n).astype(jnp.int32)

    mesh = plsc.VectorSubcoreMesh(
        core_axis_name="c", subcore_axis_name="s",
        num_cores=NC, num_subcores=NS,
    )

    @functools.partial(
        pl.kernel,
        out_type=jax.ShapeDtypeStruct((S, fg_n, (B // 128) * 1024), jnp.float32),
        mesh=mesh,
        scratch_types=[
            pltpu.VMEM((B,), jnp.int32),
            pltpu.VMEM((SUBB, D), jnp.float32),
            pltpu.VMEM((SUBB, D), jnp.float32),
            pltpu.VMEM((tlen,), jnp.float32),
            pltpu.VMEM((tlen,), jnp.float32),
            pltpu.SemaphoreType.DMA((2,)),
            pltpu.SemaphoreType.DMA((2,)),
        ],
        compiler_params=pltpu.CompilerParams(
            use_tc_tiling_on_sc=False, needs_layout_passes=False),
    )
    def run(idx_hbm, tab_hbm, out_hbm, idx_b, rows0, rows1, t0, t1, gsem, ssem):
        wid = lax.axis_index("s") * NC + lax.axis_index("c")
        iota16 = lax.iota(jnp.int32, 16)

        def transpose_to(rows, t):
            zeros = iota16 * 0
            for h in range(D // 16):
                for j in range(16):
                    fmod = (iota16 + j) & 15
                    base = iota16 * D + (fmod + 16 * h)
                    d_vec = ((fmod >> 3) + 2 * h) * tz + (fmod & 7) * 128 + iota16

                    @plsc.parallel_loop(0, SUBB, step=16, unroll=8)
                    def _(c0):
                        vals = plsc.load_gather(rows, [zeros, base + c0 * D])
                        plsc.store_scatter(t, [d_vec + c0], vals)

        def fire_gather(q, rows, sem):
            return pltpu.async_copy(
                tab_hbm.at[idx_b.at[pl.ds(q * SUBB, SUBB)]], rows, sem)

        def wait_gather(rows, sem):
            pltpu.make_async_copy(
                tab_hbm.at[idx_b.at[pl.ds(0, SUBB)]], rows, sem).wait()

        def fire_stores(t, s, q, sem):
            for fg in range(fg_n):
                pltpu.async_copy(
                    t.at[pl.ds(fg * tz, tz)],
                    out_hbm.at[s, fg, pl.ds(q * tz, tz)], sem)

        def wait_stores(t, sem):
            for fg in range(fg_n):
                pltpu.make_async_copy(
                    t.at[pl.ds(fg * tz, tz)],
                    out_hbm.at[0, fg, pl.ds(0, tz)], sem).wait()

        n_hi = S - 6 * NC * NS
        n_s = jnp.where(wid < n_hi, 7, 6)
        s0 = jnp.where(wid < n_hi, 7 * wid, 7 * n_hi + 6 * (wid - n_hi))

        if True:
            def s_body(s_i, carry):
                s = s0 + s_i
                pltpu.sync_copy(idx_hbm.at[pl.ds(s * B, B)], idx_b)
                fire_gather(0, rows0, gsem.at[0])

                def qp_body(qp, c2):
                    q0 = 2 * qp
                    fire_gather(q0 + 1, rows1, gsem.at[1])
                    wait_gather(rows0, gsem.at[0])

                    @pl.when(qp > 0)
                    def _():
                        wait_stores(t0, ssem.at[0])

                    transpose_to(rows0, t0)
                    fire_stores(t0, s, q0, ssem.at[0])

                    @pl.when(qp < n_qp - 1)
                    def _():
                        fire_gather(q0 + 2, rows0, gsem.at[0])

                    wait_gather(rows1, gsem.at[1])

                    @pl.when(qp > 0)
                    def _():
                        wait_stores(t1, ssem.at[1])

                    transpose_to(rows1, t1)
                    fire_stores(t1, s, q0 + 1, ssem.at[1])
                    return c2

                lax.fori_loop(0, n_qp, qp_body, 0)
                wait_stores(t0, ssem.at[0])
                wait_stores(t1, ssem.at[1])
                return carry

            lax.fori_loop(0, n_s, s_body, 0)

    out3 = run(idx, word_embeddings)
    out5 = out3.reshape(S, fg_n, B // 128, 8, 128)
    return out5.transpose(2, 4, 0, 1, 3).reshape(B, S, D)


# speedup vs baseline: 1.3660x; 1.3660x over previous
import functools

import numpy as np

import jax
import jax.numpy as jnp
from jax import lax
from jax.experimental import pallas as pl
from jax.experimental.pallas import tpu as pltpu
from jax.experimental.pallas import tpu_sc as plsc

NC = 2
NS = 16

SPW = 8
SUBB = 128


def kernel(sentence, word_embeddings):
    B, S = sentence.shape
    D = word_embeddings.shape[1]
    n = B * S
    n_act = S // SPW
    n_sub = B // SUBB
    n_qp = n_sub // 2
    fg_n = D // 8
    bb_sub = SUBB // 128
    tz = bb_sub * 8 * 128
    tlen = fg_n * tz

    idx = sentence.T.reshape(n).astype(jnp.int32)

    mesh = plsc.VectorSubcoreMesh(
        core_axis_name="c", subcore_axis_name="s",
        num_cores=NC, num_subcores=NS,
    )

    @functools.partial(
        pl.kernel,
        out_type=jax.ShapeDtypeStruct((S, fg_n, (B // 128) * 1024), jnp.float32),
        mesh=mesh,
        scratch_types=[
            pltpu.VMEM((B,), jnp.int32),
            pltpu.VMEM((SUBB, D), jnp.float32),
            pltpu.VMEM((SUBB, D), jnp.float32),
            pltpu.VMEM((tlen,), jnp.float32),
            pltpu.VMEM((tlen,), jnp.float32),
            pltpu.SemaphoreType.DMA((2,)),
            pltpu.SemaphoreType.DMA((2,)),
        ],
        compiler_params=pltpu.CompilerParams(
            use_tc_tiling_on_sc=False, needs_layout_passes=False),
    )
    def run(idx_hbm, tab_hbm, out_hbm, idx_b, rows0, rows1, t0, t1, gsem, ssem):
        wid = lax.axis_index("s") * NC + lax.axis_index("c")
        iota16 = lax.iota(jnp.int32, 16)

        def transpose_to(rows, t):
            zeros = iota16 * 0
            for h in range(D // 16):
                for j in range(16):
                    fmod = (iota16 + j) & 15
                    base = iota16 * D + (fmod + 16 * h)
                    d_vec = ((fmod >> 3) + 2 * h) * tz + (fmod & 7) * 128 + iota16

                    @plsc.parallel_loop(0, SUBB, step=16, unroll=4)
                    def _(c0):
                        vals = plsc.load_gather(rows, [zeros, base + c0 * D])
                        plsc.store_scatter(t, [d_vec + c0], vals)

        def fire_gather(q, rows, sem):
            return pltpu.async_copy(
                tab_hbm.at[idx_b.at[pl.ds(q * SUBB, SUBB)]], rows, sem)

        def wait_gather(rows, sem):
            pltpu.make_async_copy(
                tab_hbm.at[idx_b.at[pl.ds(0, SUBB)]], rows, sem).wait()

        def fire_stores(t, s, q, sem):
            for fg in range(fg_n):
                pltpu.async_copy(
                    t.at[pl.ds(fg * tz, tz)],
                    out_hbm.at[s, fg, pl.ds(q * tz, tz)], sem)

        def wait_stores(t, sem):
            for fg in range(fg_n):
                pltpu.make_async_copy(
                    t.at[pl.ds(fg * tz, tz)],
                    out_hbm.at[0, fg, pl.ds(0, tz)], sem).wait()

        n_hi = S - 6 * NC * NS
        n_s = jnp.where(wid < n_hi, 7, 6)
        s0 = jnp.where(wid < n_hi, 7 * wid, 7 * n_hi + 6 * (wid - n_hi))

        if True:
            def s_body(s_i, carry):
                s = s0 + s_i
                pltpu.sync_copy(idx_hbm.at[pl.ds(s * B, B)], idx_b)
                fire_gather(0, rows0, gsem.at[0])

                def qp_body(qp, c2):
                    q0 = 2 * qp
                    fire_gather(q0 + 1, rows1, gsem.at[1])
                    wait_gather(rows0, gsem.at[0])

                    @pl.when(qp > 0)
                    def _():
                        wait_stores(t0, ssem.at[0])

                    transpose_to(rows0, t0)
                    fire_stores(t0, s, q0, ssem.at[0])

                    @pl.when(qp < n_qp - 1)
                    def _():
                        fire_gather(q0 + 2, rows0, gsem.at[0])

                    wait_gather(rows1, gsem.at[1])

                    @pl.when(qp > 0)
                    def _():
                        wait_stores(t1, ssem.at[1])

                    transpose_to(rows1, t1)
                    fire_stores(t1, s, q0 + 1, ssem.at[1])
                    return c2

                lax.fori_loop(0, n_qp, qp_body, 0)
                wait_stores(t0, ssem.at[0])
                wait_stores(t1, ssem.at[1])
                return carry

            lax.fori_loop(0, n_s, s_body, 0)

    out3 = run(idx, word_embeddings)
    out5 = out3.reshape(S, fg_n, B // 128, 8, 128)
    return out5.transpose(2, 4, 0, 1, 3).reshape(B, S, D)
